# baseline (speedup 1.0000x reference)
import functools

import jax
import jax.numpy as jnp
import numpy as np
from jax import lax
from jax.experimental import pallas as pl
from jax.experimental.pallas import tpu as pltpu
from jax.experimental.pallas import tpu_sc as plsc

_NUM_LEVELS = 12
_LOG2_HASH = 19
_HASH_SIZE = 2 ** _LOG2_HASH
_BASE_RES = 16
_MAX_RES = 1024
_GROWTH = np.exp((np.log(_MAX_RES) - np.log(_BASE_RES)) / (_NUM_LEVELS - 1))
_RES = [int(_BASE_RES * _GROWTH ** i) for i in range(_NUM_LEVELS)]
_N = 1048576
_F = 2 * _NUM_LEVELS
_TBLK = _NUM_LEVELS * _HASH_SIZE // 4

_P1 = np.int32(2654435761 - (1 << 32))
_P2 = np.int32(805459861)
_MASK = np.int32(_HASH_SIZE - 1)

_NW = 32
_PTS_PER_W = _N // _NW
_C = 256
_S = _C * _NUM_LEVELS
_OW = _C * _F
_PLANE = (_N // 128) * 1024


def _sc_lookup(x0, x1, x2, table):
    mesh = plsc.VectorSubcoreMesh(core_axis_name="c", subcore_axis_name="s")

    @functools.partial(
        pl.kernel,
        mesh=mesh,
        out_type=jax.ShapeDtypeStruct((_N * _F,), jnp.float32),
        scratch_types=[
            pltpu.VMEM((_C,), jnp.float32), pltpu.VMEM((_C,), jnp.float32),
            pltpu.VMEM((_C,), jnp.float32), pltpu.VMEM((_C,), jnp.float32),
            pltpu.VMEM((_C,), jnp.float32), pltpu.VMEM((_C,), jnp.float32),
            pltpu.VMEM((_S,), jnp.int32), pltpu.VMEM((_S,), jnp.int32),
            pltpu.VMEM((_S,), jnp.int32), pltpu.VMEM((_S,), jnp.int32),
            pltpu.VMEM((_S, 8), jnp.float32), pltpu.VMEM((_S, 8), jnp.float32),
            pltpu.VMEM((_OW,), jnp.float32), pltpu.VMEM((_OW,), jnp.float32),
            pltpu.SemaphoreType.DMA,
            pltpu.SemaphoreType.DMA,
            pltpu.SemaphoreType.DMA,
        ],
        compiler_params=pltpu.CompilerParams(
            needs_layout_passes=False, use_tc_tiling_on_sc=False),
    )
    def k(x0_hbm, x1_hbm, x2_hbm, table_hbm, out_hbm,
          x0a, x1a, x2a, x0b, x1b, x2b,
          gidx_a, gidx_b, coff_a, coff_b, rows_a, rows_b, ost_a, ost_b,
          semg, semx, semw):
        wid = lax.axis_index("s") * np.int32(2) + lax.axis_index("c")
        tile_base = wid * np.int32(_PTS_PER_W)
        zero = wid * np.int32(0)
        tile_end = tile_base + np.int32(_PTS_PER_W)
        last_load = tile_end - np.int32(_C)
        lanes = lax.iota(jnp.int32, 16)

        def load_x(base, xv0, xv1, xv2):
            c0 = pltpu.async_copy(x0_hbm.at[pl.ds(base, _C)], xv0, semx)
            c1 = pltpu.async_copy(x1_hbm.at[pl.ds(base, _C)], xv1, semx)
            c2 = pltpu.async_copy(x2_hbm.at[pl.ds(base, _C)], xv2, semx)
            return (c0, c1, c2)

        def hash_chunk(loads, xv0, xv1, xv2, gidx, coff):
            for c in loads:
                c.wait()

            @pl.loop(zero, np.int32(_C), step=np.int32(16))
            def g_body(s):
                xs0 = xv0[pl.ds(s, 16)]
                xs1 = xv1[pl.ds(s, 16)]
                xs2 = xv2[pl.ds(s, 16)]
                pv = lanes + s
                for i in range(_NUM_LEVELS):
                    r = jnp.float32(_RES[i])
                    a0 = (xs0 * r).astype(jnp.int32)
                    a1 = (xs1 * r).astype(jnp.int32)
                    a2 = (xs2 * r).astype(jnp.int32)
                    h = (a0 ^ (a1 * _P1) ^ (a2 * _P2)) & _MASK
                    blk = (h >> np.int32(2)) | np.int32(i << 17)
                    off = (h & np.int32(3)) << np.int32(1)
                    pos = pv + np.int32(i * _C)
                    plsc.store_scatter(gidx, [pos], blk)
                    plsc.store_scatter(coff, [pos], off)

        def compact(rows_v, coff, ost):
            @pl.loop(zero, np.int32(_S), step=np.int32(16))
            def c_body(g):
                offv = coff[pl.ds(g, 16)]
                rowv = lanes + g
                v0 = plsc.load_gather(rows_v, [rowv, offv])
                v1 = plsc.load_gather(rows_v, [rowv, offv + np.int32(1)])
                p0 = ((g >> np.int32(10)) << np.int32(11)) \
                    | ((g >> np.int32(7)) & np.int32(1)) << np.int32(10) \
                    | ((g >> np.int32(8)) & np.int32(3)) << np.int32(8) \
                    | (g & np.int32(127))
                posv = lanes + p0
                plsc.store_scatter(ost, [posv], v0)
                plsc.store_scatter(ost, [posv + np.int32(128)], v1)

        def write_descs(base, ost):
            return [
                pltpu.make_async_copy(
                    ost.at[pl.ds(np.int32(tr * 2048), 2048)],
                    out_hbm.at[pl.ds(base * np.int32(8)
                                     + np.int32(tr * _PLANE), 2048)],
                    semw)
                for tr in range(3)]

        def issue_writes(base, ost):
            for d in write_descs(base, ost):
                d.start()

        def wait_writes(base, ost):
            for d in write_descs(base, ost):
                d.wait()

        hash_chunk(load_x(tile_base, x0a, x1a, x2a), x0a, x1a, x2a,
                   gidx_a, coff_a)

        @pl.loop(tile_base, tile_end, step=np.int32(2 * _C))
        def chunk_pair(base):
            not_first = base > tile_base

            @pl.when(not_first)
            def _():
                wait_writes(base - np.int32(2 * _C), ost_a)

            nb = base + np.int32(_C)
            lb = load_x(nb, x0b, x1b, x2b)
            cpa = pltpu.async_copy(table_hbm.at[gidx_a], rows_a, semg)
            hash_chunk(lb, x0b, x1b, x2b, gidx_b, coff_b)
            cpa.wait()

            @pl.when(not_first)
            def _():
                wait_writes(base - np.int32(_C), ost_b)

            na = jnp.minimum(base + np.int32(2 * _C), last_load)
            la = load_x(na, x0a, x1a, x2a)
            cpb = pltpu.async_copy(table_hbm.at[gidx_b], rows_b, semg)
            compact(rows_a, coff_a, ost_a)
            issue_writes(base, ost_a)
            hash_chunk(la, x0a, x1a, x2a, gidx_a, coff_a)
            cpb.wait()
            compact(rows_b, coff_b, ost_b)
            issue_writes(nb, ost_b)

        wait_writes(tile_end - np.int32(2 * _C), ost_a)
        wait_writes(tile_end - np.int32(_C), ost_b)

    return k(x0, x1, x2, table)


def kernel(x, embeddings):
    x = x.astype(jnp.float32)
    tw = (embeddings.astype(jnp.float32)
          .reshape(_TBLK, 8))
    out1d = _sc_lookup(x[:, 0], x[:, 1], x[:, 2], tw)
    return (out1d.reshape(_F // 8, _N // 128, 8, 128)
            .transpose(1, 3, 0, 2)
            .reshape(_N, _F))
